# baseline (speedup 1.0000x reference)
import math

import jax
import jax.numpy as jnp
from jax.experimental import pallas as pl

_LAT = 721
_LON = 1440
_LEVEL = 8
_RES = 0.25


def _upsample_bilinear(g, H, W):
    h, w = g.shape
    ys = jnp.maximum((jnp.arange(H, dtype=jnp.float32) + 0.5) * (h / H) - 0.5, 0.0)
    xs = jnp.maximum((jnp.arange(W, dtype=jnp.float32) + 0.5) * (w / W) - 0.5, 0.0)
    y0 = jnp.floor(ys).astype(jnp.int32)
    x0 = jnp.floor(xs).astype(jnp.int32)
    y1 = jnp.minimum(y0 + 1, h - 1)
    x1 = jnp.minimum(x0 + 1, w - 1)
    wy = (ys - y0.astype(jnp.float32))[:, None]
    wx = (xs - x0.astype(jnp.float32))[None, :]
    gy0 = g[y0]
    gy1 = g[y1]
    top = gy0[:, x0] * (1.0 - wx) + gy0[:, x1] * wx
    bot = gy1[:, x0] * (1.0 - wx) + gy1[:, x1] * wx
    return top * (1.0 - wy) + bot * wy


def _bilin(grid, lat_idx, lon_idx):
    lat_floor = jnp.clip(jnp.floor(lat_idx).astype(jnp.int32), 0, grid.shape[0] - 1)
    lon_floor = jnp.clip(jnp.floor(lon_idx).astype(jnp.int32), 0, grid.shape[1] - 1)
    lat_ceil = jnp.clip(lat_floor + 1, 0, grid.shape[0] - 1)
    lon_ceil = jnp.clip(lon_floor + 1, 0, grid.shape[1] - 1)
    vff = grid[lat_floor, lon_floor]
    vfc = grid[lat_floor, lon_ceil]
    vcf = grid[lat_ceil, lon_floor]
    vcc = grid[lat_ceil, lon_ceil]
    fy = lat_idx - lat_floor.astype(jnp.float32)
    fx = lon_idx - lon_floor.astype(jnp.float32)
    vf = vff + fx * (vfc - vff)
    vc = vcf + fx * (vcc - vcf)
    return vf + fy * (vc - vf)


def _copy_kernel(x_ref, o_ref):
    o_ref[...] = x_ref[...]


def kernel(x, grid_0, grid_1, grid_2, grid_3, grid_4, grid_5, grid_6, grid_7):
    grids = [grid_0, grid_1, grid_2, grid_3, grid_4, grid_5, grid_6, grid_7]
    ups = [_upsample_bilinear(g[0, 0], _LAT, _LON) for g in grids]
    lat = x[..., 0:1]
    lon = x[..., 1:2]
    lat_idx = (90.0 - lat) / _RES
    lon_idx = lon / _RES
    reps = [jnp.squeeze(_bilin(up, lat_idx, lon_idx)) for up in ups]
    out = jnp.stack(reps, axis=-1)
    return pl.pallas_call(
        _copy_kernel,
        grid=(250,),
        in_specs=[pl.BlockSpec((4000, 8), lambda i: (i, 0))],
        out_specs=pl.BlockSpec((4000, 8), lambda i: (i, 0)),
        out_shape=jax.ShapeDtypeStruct(out.shape, out.dtype),
    )(out)


# speedup vs baseline: 1.7663x; 1.7663x over previous
import functools
import math

import jax
import jax.numpy as jnp
from jax import lax
from jax.experimental import pallas as pl
from jax.experimental.pallas import tpu as pltpu
from jax.experimental.pallas import tpu_sc as plsc

_LAT = 721
_LON = 1440
_LEVEL = 8
_N = 1000000

_NW = 32
_CHUNK = 1024
_CPW = 31
_PPW = _CHUNK * _CPW
_NPAD = _PPW * _NW
_GROUPS = _CHUNK // 16


def _interp_matrix(h, H):
    dst = lax.broadcasted_iota(jnp.int32, (H, h), 0).astype(jnp.float32)
    src = lax.broadcasted_iota(jnp.int32, (H, h), 1)
    ys = jnp.maximum((dst + 0.5) * (h / H) - 0.5, 0.0)
    y0 = jnp.floor(ys).astype(jnp.int32)
    y1 = jnp.minimum(y0 + 1, h - 1)
    fy = ys - y0.astype(jnp.float32)
    return jnp.where(src == y0, 1.0 - fy, 0.0) + jnp.where(src == y1, fy, 0.0)


def _upsample_body(g_ref, o_ref, *, h, w):
    g = g_ref[...]
    cxt = _interp_matrix(w, _LON).T
    ry = _interp_matrix(h, _LAT)
    m = jnp.dot(g, cxt, preferred_element_type=jnp.float32)
    o_ref[...] = jnp.dot(ry, m, preferred_element_type=jnp.float32)


def _upsample_level(g2d):
    h, w = g2d.shape
    return pl.pallas_call(
        functools.partial(_upsample_body, h=h, w=w),
        out_shape=jax.ShapeDtypeStruct((_LAT, _LON), jnp.float32),
    )(g2d)


def _sc_sample_body(t_hbm, lat_hbm, lon_hbm, out_hbm,
                    latv, lonv, idx0, idx1, wyb, wxb,
                    rows0, rows1, tmp, outb, sem):
    nc = 2
    wid = lax.axis_index("s") * nc + lax.axis_index("c")
    base = wid * _PPW

    def chunk_body(ci, _):
        pbase = base + ci * _CHUNK
        pltpu.sync_copy(lat_hbm.at[pl.ds(pbase, _CHUNK)], latv)
        pltpu.sync_copy(lon_hbm.at[pl.ds(pbase, _CHUNK)], lonv)

        def index_body(g, _):
            la = latv[pl.ds(g * 16, 16)]
            lo = lonv[pl.ds(g * 16, 16)]
            y = (90.0 - la) * 4.0
            x = lo * 4.0
            y0 = jnp.clip(y.astype(jnp.int32), 0, _LAT - 1)
            x0 = jnp.clip(x.astype(jnp.int32), 0, _LON - 1)
            wyb[pl.ds(g * 16, 16)] = y - y0.astype(jnp.float32)
            wxb[pl.ds(g * 16, 16)] = x - x0.astype(jnp.float32)
            y1 = jnp.minimum(y0 + 1, _LAT - 1)
            idx0[pl.ds(g * 16, 16)] = y0 * _LON + x0
            idx1[pl.ds(g * 16, 16)] = y1 * _LON + x0
            return 0

        lax.fori_loop(0, _GROUPS, index_body, 0, unroll=False)

        copies = []
        for j in range(_CHUNK // 128):
            copies.append(pltpu.async_copy(
                t_hbm.at[idx0.at[pl.ds(j * 128, 128)]],
                rows0.at[pl.ds(j * 128, 128), :], sem))
            copies.append(pltpu.async_copy(
                t_hbm.at[idx1.at[pl.ds(j * 128, 128)]],
                rows1.at[pl.ds(j * 128, 128), :], sem))
        for cp in copies:
            cp.wait()

        def combine_body(g, _):
            wy16 = wyb[pl.ds(g * 16, 16)]
            wx16 = wxb[pl.ds(g * 16, 16)]
            for j in range(16):
                p = g * 16 + j
                va0 = rows0[p, :]
                va1 = rows1[p, :]
                by = jnp.full((16,), wy16[j], jnp.float32)
                bx = jnp.full((16,), wx16[j], jnp.float32)
                m = va0 + by * (va1 - va0)
                tmp[pl.ds(j * 32, 16)] = m
                msh = tmp[pl.ds(j * 32 + 8, 16)]
                o = m + bx * (msh - m)
                outb[pl.ds(p * 8, 16)] = o
            return 0

        lax.fori_loop(0, _GROUPS, combine_body, 0, unroll=False)

        pltpu.sync_copy(outb.at[pl.ds(0, _CHUNK * _LEVEL)],
                        out_hbm.at[pl.ds(pbase * _LEVEL, _CHUNK * _LEVEL)])
        return 0

    lax.fori_loop(0, _CPW, chunk_body, 0, unroll=False)


def _sc_sample(table, lat, lon):
    mesh = plsc.VectorSubcoreMesh(core_axis_name="c", subcore_axis_name="s")
    f = pl.kernel(
        _sc_sample_body,
        out_type=jax.ShapeDtypeStruct((_NPAD * _LEVEL,), jnp.float32),
        mesh=mesh,
        compiler_params=pltpu.CompilerParams(use_tc_tiling_on_sc=False),
        scratch_types=[
            pltpu.VMEM((_CHUNK,), jnp.float32),
            pltpu.VMEM((_CHUNK,), jnp.float32),
            pltpu.VMEM((_CHUNK,), jnp.int32),
            pltpu.VMEM((_CHUNK,), jnp.int32),
            pltpu.VMEM((_CHUNK,), jnp.float32),
            pltpu.VMEM((_CHUNK,), jnp.float32),
            pltpu.VMEM((_CHUNK, 16), jnp.float32),
            pltpu.VMEM((_CHUNK, 16), jnp.float32),
            pltpu.VMEM((512,), jnp.float32),
            pltpu.VMEM((_CHUNK * _LEVEL + 16,), jnp.float32),
            pltpu.SemaphoreType.DMA,
        ],
    )
    return f(table, lat, lon)


def kernel(x, grid_0, grid_1, grid_2, grid_3, grid_4, grid_5, grid_6, grid_7):
    grids = [grid_0, grid_1, grid_2, grid_3, grid_4, grid_5, grid_6, grid_7]
    ups = [grids[0][0, 0]]
    for g in grids[1:]:
        ups.append(_upsample_level(g[0, 0]))
    w = jnp.stack(ups)
    ws = jnp.concatenate([w[:, :, 1:], w[:, :, -1:]], axis=2)
    t16 = jnp.concatenate([w, ws], axis=0)
    table = t16.transpose(1, 2, 0).reshape(_LAT * _LON, 16)

    lat = jnp.pad(x[:, 0], (0, _NPAD - _N))
    lon = jnp.pad(x[:, 1], (0, _NPAD - _N))
    flat = _sc_sample(table, lat, lon)
    return flat.reshape(_NPAD, _LEVEL)[:_N]


# speedup vs baseline: 2.0373x; 1.1534x over previous
import functools
import math

import jax
import jax.numpy as jnp
from jax import lax
from jax.experimental import pallas as pl
from jax.experimental.pallas import tpu as pltpu
from jax.experimental.pallas import tpu_sc as plsc

_LAT = 721
_LON = 1440
_LEVEL = 8
_N = 1000000

_NW = 32
_CHUNK = 1024
_CPW = 31
_PPW = _CHUNK * _CPW
_NPAD = _PPW * _NW
_GROUPS = _CHUNK // 16


_LONP = 1536


def _interp_matrix(h, H, HP=None):
    HP = H if HP is None else HP
    dsti = lax.broadcasted_iota(jnp.int32, (HP, h), 0)
    dst = dsti.astype(jnp.float32)
    src = lax.broadcasted_iota(jnp.int32, (HP, h), 1)
    ys = jnp.maximum((dst + 0.5) * (h / H) - 0.5, 0.0)
    y0 = jnp.floor(ys).astype(jnp.int32)
    y1 = jnp.minimum(y0 + 1, h - 1)
    fy = ys - y0.astype(jnp.float32)
    w = jnp.where(src == y0, 1.0 - fy, 0.0) + jnp.where(src == y1, fy, 0.0)
    return jnp.where(dsti < H, w, 0.0)


def _upsample_body(g_ref, o_ref, *, h, w):
    g = g_ref[...]
    cxt = _interp_matrix(w, _LON, _LONP).T
    ry = _interp_matrix(h, _LAT)
    m = jnp.dot(g, cxt, preferred_element_type=jnp.float32)
    o_ref[...] = jnp.dot(ry, m, preferred_element_type=jnp.float32)


def _upsample_level(g2d):
    h, w = g2d.shape
    return pl.pallas_call(
        functools.partial(_upsample_body, h=h, w=w),
        out_shape=jax.ShapeDtypeStruct((_LAT, _LONP), jnp.float32),
    )(g2d)


def _interleave_body(*refs):
    us = refs[:_LEVEL]
    o_ref = refs[_LEVEL]
    xb = pl.program_id(0)
    w128 = pl.multiple_of((xb // 8) * 128, 128)
    w128b = pl.multiple_of(jnp.minimum(w128 + 128, _LONP - 128), 128)
    o = (xb % 8) * 16
    xl = lax.broadcasted_iota(jnp.int32, (17, 256), 1) // 16
    l = lax.broadcasted_iota(jnp.int32, (17, 256), 1) % 16
    dxr = lax.broadcasted_iota(jnp.int32, (17, 256), 0)
    shift = (l >= 8).astype(jnp.int32)
    lev = jnp.where(l < 8, l, 15 - l)
    dx = jnp.minimum(xb * 16 + xl + shift, _LON - 1) - xb * 16
    acc = jnp.zeros((_LAT, 256), jnp.float32)
    for i in range(_LEVEL):
        wm = jnp.where((dxr == dx) & (lev == i), 1.0, 0.0)
        wina = us[i][:, pl.ds(w128, 128)]
        rolled = pltpu.roll(wina, -o, axis=1)
        winb = us[i][:, pl.ds(w128b, 128)]
        col17 = jnp.where(o == 112, winb[:, 0:1], rolled[:, 16:17])
        b = jnp.concatenate([rolled[:, :16], col17], axis=1)
        acc = acc + jnp.dot(b, wm, preferred_element_type=jnp.float32)
    o_ref[0] = acc


def _interleave(ups):
    return pl.pallas_call(
        _interleave_body,
        grid=(_LON // 16,),
        in_specs=[pl.BlockSpec((_LAT, _LONP), lambda i: (0, 0))] * _LEVEL,
        out_specs=pl.BlockSpec((1, _LAT, 256), lambda i: (i, 0, 0)),
        out_shape=jax.ShapeDtypeStruct((_LON // 16, _LAT, 256), jnp.float32),
    )(*ups)


def _sc_sample_body(t_hbm, lat_hbm, lon_hbm, out_hbm,
                    latv, lonv, idx0, idx1, wyb, wxb,
                    rows0, rows1, outb, sem):
    nc = 2
    wid = lax.axis_index("s") * nc + lax.axis_index("c")
    base = wid * _PPW

    def chunk_body(ci, _):
        pbase = base + ci * _CHUNK
        pltpu.sync_copy(lat_hbm.at[pl.ds(pbase, _CHUNK)], latv)
        pltpu.sync_copy(lon_hbm.at[pl.ds(pbase, _CHUNK)], lonv)

        def index_body(g, _):
            la = latv[pl.ds(g * 16, 16)]
            lo = lonv[pl.ds(g * 16, 16)]
            y = (90.0 - la) * 4.0
            x = lo * 4.0
            y0 = jnp.clip(y.astype(jnp.int32), 0, _LAT - 1)
            x0 = jnp.clip(x.astype(jnp.int32), 0, _LON - 1)
            wyb[pl.ds(g * 16, 16)] = y - y0.astype(jnp.float32)
            wxb[pl.ds(g * 16, 16)] = x - x0.astype(jnp.float32)
            y1 = jnp.minimum(y0 + 1, _LAT - 1)
            xblk = (x0 >> 4) * (_LAT * 16) + (x0 & 15)
            idx0[pl.ds(g * 16, 16)] = xblk + y0 * 16
            idx1[pl.ds(g * 16, 16)] = xblk + y1 * 16
            return 0

        lax.fori_loop(0, _GROUPS, index_body, 0, unroll=False)

        copies = []
        for j in range(_CHUNK // 128):
            copies.append(pltpu.async_copy(
                t_hbm.at[idx0.at[pl.ds(j * 128, 128)]],
                rows0.at[pl.ds(j * 128, 128), :], sem))
            copies.append(pltpu.async_copy(
                t_hbm.at[idx1.at[pl.ds(j * 128, 128)]],
                rows1.at[pl.ds(j * 128, 128), :], sem))
        for cp in copies:
            cp.wait()

        def combine_body(g, _):
            wy16 = wyb[pl.ds(g * 16, 16)]
            wx16 = wxb[pl.ds(g * 16, 16)]
            for j in range(16):
                p = g * 16 + j
                va0 = rows0[p, :]
                va1 = rows1[p, :]
                by = jnp.full((16,), wy16[j], jnp.float32)
                bx = jnp.full((16,), wx16[j], jnp.float32)
                m = va0 + by * (va1 - va0)
                mr = lax.rev(m, (0,))
                o = m + bx * (mr - m)
                outb[pl.ds(p * 8, 16)] = o
            return 0

        lax.fori_loop(0, _GROUPS, combine_body, 0, unroll=False)

        pltpu.sync_copy(outb.at[pl.ds(0, _CHUNK * _LEVEL)],
                        out_hbm.at[pl.ds(pbase * _LEVEL, _CHUNK * _LEVEL)])
        return 0

    lax.fori_loop(0, _CPW, chunk_body, 0, unroll=False)


def _sc_sample(table, lat, lon):
    mesh = plsc.VectorSubcoreMesh(core_axis_name="c", subcore_axis_name="s")
    f = pl.kernel(
        _sc_sample_body,
        out_type=jax.ShapeDtypeStruct((_NPAD * _LEVEL,), jnp.float32),
        mesh=mesh,
        compiler_params=pltpu.CompilerParams(use_tc_tiling_on_sc=False),
        scratch_types=[
            pltpu.VMEM((_CHUNK,), jnp.float32),
            pltpu.VMEM((_CHUNK,), jnp.float32),
            pltpu.VMEM((_CHUNK,), jnp.int32),
            pltpu.VMEM((_CHUNK,), jnp.int32),
            pltpu.VMEM((_CHUNK,), jnp.float32),
            pltpu.VMEM((_CHUNK,), jnp.float32),
            pltpu.VMEM((_CHUNK, 16), jnp.float32),
            pltpu.VMEM((_CHUNK, 16), jnp.float32),
            pltpu.VMEM((_CHUNK * _LEVEL + 16,), jnp.float32),
            pltpu.SemaphoreType.DMA,
        ],
    )
    return f(table, lat, lon)


def kernel(x, grid_0, grid_1, grid_2, grid_3, grid_4, grid_5, grid_6, grid_7):
    grids = [grid_0, grid_1, grid_2, grid_3, grid_4, grid_5, grid_6, grid_7]
    ups = [jnp.pad(grids[0][0, 0], ((0, 0), (0, _LONP - _LON)))]
    for g in grids[1:]:
        ups.append(_upsample_level(g[0, 0]))
    table = _interleave(ups).reshape(_LAT * _LON, 16)

    lat = jnp.pad(x[:, 0], (0, _NPAD - _N))
    lon = jnp.pad(x[:, 1], (0, _NPAD - _N))
    flat = _sc_sample(table, lat, lon)
    return flat.reshape(_NPAD, _LEVEL)[:_N]
